# baseline (speedup 1.0000x reference)
import functools

import jax
import jax.numpy as jnp
from jax import lax
from jax.experimental import pallas as pl
from jax.experimental.pallas import tpu as pltpu
from jax.experimental.pallas import tpu_sc as plsc

B, C, N, K = 4, 64, 10000, 16
NK = N * K
NSC = 32
IW = 128
RPC = 4
NROWS = B * NK // IW
NCHUNKS = NROWS // RPC
CHUNK = RPC * IW

_sc_mesh = plsc.VectorSubcoreMesh(core_axis_name="c", subcore_axis_name="s")


@functools.partial(
    pl.kernel,
    mesh=_sc_mesh,
    compiler_params=pltpu.CompilerParams(
        use_tc_tiling_on_sc=False, needs_layout_passes=False
    ),
    out_type=jax.ShapeDtypeStruct((B * NK, 2 * C), jnp.float32),
    scratch_types=[
        pltpu.VMEM((RPC, IW), jnp.int32),
        pltpu.VMEM((CHUNK, C), jnp.float32),
        pltpu.SemaphoreType.DMA,
        pltpu.SemaphoreType.DMA,
    ],
)
def _sc_assemble(ftab_hbm, idx_hbm, rpe_hbm, out_hbm, idx_buf, rows_buf, sem, sem2):
    wid = lax.axis_index("s") * 2 + lax.axis_index("c")

    def step(t, carry):
        chunk_id = wid + NSC * t

        @pl.when(chunk_id < NCHUNKS)
        def _():
            r0 = chunk_id * RPC
            j0 = chunk_id * CHUNK
            rpe_cp = pltpu.async_copy(
                rpe_hbm.at[pl.ds(j0, CHUNK), :],
                out_hbm.at[pl.ds(j0, CHUNK), pl.ds(0, C)],
                sem2,
            )
            pltpu.sync_copy(idx_hbm.at[pl.ds(r0, RPC), :], idx_buf)
            cps = [
                pltpu.async_copy(
                    ftab_hbm.at[idx_buf.at[r]],
                    rows_buf.at[pl.ds(r * IW, IW), :],
                    sem,
                )
                for r in range(RPC)
            ]
            for cp in cps:
                cp.wait()
            pltpu.sync_copy(
                rows_buf, out_hbm.at[pl.ds(j0, CHUNK), pl.ds(C, C)]
            )
            rpe_cp.wait()

        return carry

    lax.fori_loop(0, (NCHUNKS + NSC - 1) // NSC, step, 0)


def kernel(relative_position_encoding, features, neighbors):
    ftab = jnp.transpose(features[:, :, :, 0], (0, 2, 1)).reshape(B * N, C)
    rpe_t = jnp.transpose(relative_position_encoding, (0, 2, 3, 1)).reshape(
        B * NK, C
    )
    idxg = neighbors + (jnp.arange(B, dtype=jnp.int32) * N)[:, None, None]
    idxg = idxg.reshape(NROWS, IW)
    out = _sc_assemble(ftab, idxg, rpe_t)
    return jnp.transpose(out.reshape(B, N, K, 2 * C), (0, 3, 1, 2))


# speedup vs baseline: 7.0301x; 7.0301x over previous
import functools

import jax
import jax.numpy as jnp
from jax import lax
from jax.experimental import pallas as pl
from jax.experimental.pallas import tpu as pltpu
from jax.experimental.pallas import tpu_sc as plsc

B, C, N, K = 4, 64, 10000, 16
NK = N * K
NSC = 32
IW = 128
RPC = 4
NROWS = B * NK // IW
NCHUNKS = NROWS // RPC
CHUNK = RPC * IW

_sc_mesh = plsc.VectorSubcoreMesh(core_axis_name="c", subcore_axis_name="s")


@functools.partial(
    pl.kernel,
    mesh=_sc_mesh,
    compiler_params=pltpu.CompilerParams(
        use_tc_tiling_on_sc=False, needs_layout_passes=False
    ),
    out_type=jax.ShapeDtypeStruct((B * NK, 2 * C), jnp.float32),
    scratch_types=[
        pltpu.VMEM((RPC, IW), jnp.int32),
        pltpu.VMEM((CHUNK, 2 * C), jnp.float32),
        pltpu.SemaphoreType.DMA,
    ],
)
def _sc_gather(ftab_hbm, idx_hbm, gath_hbm, idx_buf, rows_buf, sem):
    wid = lax.axis_index("s") * 2 + lax.axis_index("c")

    def step(t, carry):
        chunk_id = wid + NSC * t

        @pl.when(chunk_id < NCHUNKS)
        def _():
            r0 = chunk_id * RPC
            pltpu.sync_copy(idx_hbm.at[pl.ds(r0, RPC), :], idx_buf)
            cps = [
                pltpu.async_copy(
                    ftab_hbm.at[idx_buf.at[r]],
                    rows_buf.at[pl.ds(r * IW, IW), :],
                    sem,
                )
                for r in range(RPC)
            ]
            for cp in cps:
                cp.wait()
            pltpu.sync_copy(
                rows_buf, gath_hbm.at[pl.ds(chunk_id * CHUNK, CHUNK), :]
            )

        return carry

    lax.fori_loop(0, (NCHUNKS + NSC - 1) // NSC, step, 0)


_JB = 8000


def _concat_body(rpe_ref, gath_ref, out_ref):
    out_ref[0] = jnp.concatenate(
        [rpe_ref[0], gath_ref[0][:, 0:C]], axis=1
    )


def _tc_concat(rpe_t, gath2):
    return pl.pallas_call(
        _concat_body,
        grid=(B, NK // _JB),
        in_specs=[
            pl.BlockSpec((1, _JB, C), lambda b, j: (b, j, 0)),
            pl.BlockSpec((1, _JB, 2 * C), lambda b, j: (b, j, 0)),
        ],
        out_specs=pl.BlockSpec((1, _JB, 2 * C), lambda b, j: (b, j, 0)),
        out_shape=jax.ShapeDtypeStruct((B, NK, 2 * C), jnp.float32),
    )(rpe_t, gath2)


def kernel(relative_position_encoding, features, neighbors):
    ftab = jnp.transpose(features[:, :, :, 0], (0, 2, 1)).reshape(B * N, C)
    ftab = jnp.pad(ftab, ((0, 0), (0, C)))
    rpe_t = jnp.transpose(relative_position_encoding, (0, 2, 3, 1)).reshape(
        B, NK, C
    )
    idxg = neighbors + (jnp.arange(B, dtype=jnp.int32) * N)[:, None, None]
    idxg = idxg.reshape(NROWS, IW)
    gath2 = _sc_gather(ftab, idxg).reshape(B, NK, 2 * C)
    out = _tc_concat(rpe_t, gath2)
    return jnp.transpose(out.reshape(B, N, K, 2 * C), (0, 3, 1, 2))
